# baseline (speedup 1.0000x reference)
import functools

import jax
import jax.numpy as jnp
from jax import lax
from jax.experimental import pallas as pl
from jax.experimental.pallas import tpu as pltpu
from jax.experimental.pallas import tpu_sc as plsc

VOCAB = 100000
HIDDEN = 128
B = 1024
L = 50

NC = 2
NS = 16
NW = NC * NS
LANES = 16

TOTAL = B * L
PER_W = TOTAL // NW
SEQ_PER_CHUNK = 8
CHUNK = SEQ_PER_CHUNK * L
N_CHUNKS = PER_W // CHUNK


def _sc_embed(ids_flat, token_embedding, pos_block):
    mesh = plsc.VectorSubcoreMesh(core_axis_name="c", subcore_axis_name="s")

    @functools.partial(
        pl.kernel,
        out_type=jax.ShapeDtypeStruct((TOTAL, HIDDEN), jnp.float32),
        mesh=mesh,
        scratch_types=[
            pltpu.VMEM((CHUNK,), jnp.int32),
            pltpu.VMEM((CHUNK, HIDDEN), jnp.float32),
            pltpu.VMEM((L, HIDDEN), jnp.float32),
            pltpu.SemaphoreType.DMA,
        ],
    )
    def k(ids_hbm, table_hbm, pos_hbm, out_hbm, idx_v, rows_v, pos_v, sem):
        wid = lax.axis_index("s") * NC + lax.axis_index("c")
        pltpu.sync_copy(pos_hbm, pos_v)

        @pl.loop(0, N_CHUNKS)
        def _(g):
            base = wid * PER_W + g * CHUNK
            pltpu.sync_copy(ids_hbm.at[pl.ds(base, CHUNK)], idx_v)
            pltpu.async_copy(table_hbm.at[idx_v], rows_v, sem).wait()

            @pl.loop(0, SEQ_PER_CHUNK)
            def _(s):
                @pl.loop(0, L)
                def _(l):
                    row = s * L + l
                    for c1 in range(0, HIDDEN, LANES):
                        slc = (pl.ds(row, 1), pl.ds(c1, LANES))
                        pslc = (pl.ds(l, 1), pl.ds(c1, LANES))
                        rows_v.at[*slc][...] = (
                            rows_v.at[*slc][...] + pos_v.at[*pslc][...]
                        )

            pltpu.sync_copy(rows_v, out_hbm.at[pl.ds(base, CHUNK)])

    return k(ids_flat, token_embedding, pos_block)


def kernel(input_ids, token_embedding, position_embedding):
    ids_flat = input_ids.reshape(TOTAL).astype(jnp.int32)
    pos_block = position_embedding[0, :L, :]
    out = _sc_embed(ids_flat, token_embedding, pos_block)
    return out.reshape(B, L, HIDDEN)


# speedup vs baseline: 2.4555x; 2.4555x over previous
import functools

import jax
import jax.numpy as jnp
from jax import lax
from jax.experimental import pallas as pl
from jax.experimental.pallas import tpu as pltpu
from jax.experimental.pallas import tpu_sc as plsc

VOCAB = 100000
HIDDEN = 128
B = 1024
L = 50

NC = 2
NS = 16
NW = NC * NS
LANES = 16

TOTAL = B * L
PER_W = TOTAL // NW
SEQ_PER_CHUNK = 8
CHUNK = SEQ_PER_CHUNK * L
N_CHUNKS = PER_W // CHUNK


def _sc_embed(ids_flat, token_embedding, pos_block):
    mesh = plsc.VectorSubcoreMesh(core_axis_name="c", subcore_axis_name="s")

    @functools.partial(
        pl.kernel,
        out_type=jax.ShapeDtypeStruct((B, L, HIDDEN), jnp.float32),
        mesh=mesh,
        scratch_types=[
            pltpu.VMEM((PER_W,), jnp.int32),
            pltpu.VMEM((CHUNK, HIDDEN), jnp.float32),
            pltpu.VMEM((CHUNK, HIDDEN), jnp.float32),
            pltpu.VMEM((L, HIDDEN), jnp.float32),
            pltpu.SemaphoreType.DMA,
            pltpu.SemaphoreType.DMA,
            pltpu.SemaphoreType.DMA,
            pltpu.SemaphoreType.DMA,
        ],
    )
    def k(ids_hbm, table_hbm, pos_hbm, out_hbm,
          idx_v, rows0, rows1, pos_v, gsem0, gsem1, osem0, osem1):
        wid = lax.axis_index("s") * NC + lax.axis_index("c")
        base = wid * PER_W
        pltpu.sync_copy(ids_hbm.at[pl.ds(base, PER_W)], idx_v)
        pltpu.sync_copy(pos_hbm, pos_v)

        rows = (rows0, rows1)
        gsems = (gsem0, gsem1)
        osems = (osem0, osem1)
        seq_base = wid * (PER_W // L)

        def add_pos(rv):
            @pl.loop(0, L)
            def _(l):
                for c1 in range(0, HIDDEN, LANES):
                    p = pos_v.at[pl.ds(l, 1), pl.ds(c1, LANES)][...]
                    for s in range(SEQ_PER_CHUNK):
                        slc = (pl.ds(s * L + l, 1), pl.ds(c1, LANES))
                        rv.at[*slc][...] = rv.at[*slc][...] + p

        def start_gather(g):
            return pltpu.async_copy(
                table_hbm.at[idx_v.at[pl.ds(g * CHUNK, CHUNK)]],
                rows[g % 2], gsems[g % 2])

        def start_out(g):
            rv = rows[g % 2]
            cps = []
            for s in range(SEQ_PER_CHUNK):
                cps.append(pltpu.async_copy(
                    rv.at[pl.ds(s * L, L)],
                    out_hbm.at[seq_base + g * SEQ_PER_CHUNK + s],
                    osems[g % 2]))
            return cps

        gcp = [None] * N_CHUNKS
        ocp = [None] * N_CHUNKS
        gcp[0] = start_gather(0)
        for g in range(N_CHUNKS):
            if g + 1 < N_CHUNKS:
                if g + 1 >= 2:
                    for cp in ocp[g - 1]:
                        cp.wait()
                gcp[g + 1] = start_gather(g + 1)
            gcp[g].wait()
            add_pos(rows[g % 2])
            ocp[g] = start_out(g)
        for cp in ocp[N_CHUNKS - 2]:
            cp.wait()
        for cp in ocp[N_CHUNKS - 1]:
            cp.wait()

    return k(ids_flat, token_embedding, pos_block)


def kernel(input_ids, token_embedding, position_embedding):
    ids_flat = input_ids.reshape(TOTAL).astype(jnp.int32)
    pos_block = position_embedding[0, :L, :]
    return _sc_embed(ids_flat, token_embedding, pos_block)
